# initial kernel scaffold (unmeasured)
import jax
import jax.numpy as jnp
from jax import lax
from jax.experimental import pallas as pl
from jax.experimental.pallas import tpu as pltpu

N_DEV = 4


def kernel(x, w_mat, scale_x, scale_w):
    m_per, k = x.shape
    _, n = w_mat.shape
    n_per = n // N_DEV

    def body(x_ref, w_ref, sx_ref, sw_ref, out_ref,
             y_ref, comm_ref, send_sems, recv_sems):
        my = lax.axis_index("i")

        barrier_sem = pltpu.get_barrier_semaphore()
        for p in range(1, N_DEV):
            pl.semaphore_signal(
                barrier_sem, inc=1,
                device_id=((my + p) % N_DEV,),
                device_id_type=pl.DeviceIdType.MESH,
            )
        pl.semaphore_wait(barrier_sem, N_DEV - 1)

        xb = x_ref[...].astype(jnp.bfloat16)
        wb = w_ref[...].astype(jnp.bfloat16)
        acc = jnp.dot(xb, wb, preferred_element_type=jnp.float32)
        sc = sx_ref[0] * sw_ref[0]
        yv = acc * sc
        yv = yv / (1.0 + jnp.exp(-jnp.clip(yv, -60.0, 60.0)))

        for d in range(N_DEV):
            y_ref[d] = yv[:, d * n_per:(d + 1) * n_per]

        out_ref[pl.ds(my * m_per, m_per), :] = lax.dynamic_slice(
            yv, (0, my * n_per), (m_per, n_per))

        sends = []
        for p in range(1, N_DEV):
            dst = (my + p) % N_DEV
            rdma = pltpu.make_async_remote_copy(
                src_ref=y_ref.at[dst],
                dst_ref=comm_ref.at[p - 1],
                send_sem=send_sems.at[p - 1],
                recv_sem=recv_sems.at[p - 1],
                device_id=(dst,),
                device_id_type=pl.DeviceIdType.MESH,
            )
            rdma.start()
            sends.append(rdma)

        for p in range(1, N_DEV):
            src_id = (my - p) % N_DEV
            recv = pltpu.make_async_remote_copy(
                src_ref=y_ref.at[0],
                dst_ref=comm_ref.at[p - 1],
                send_sem=send_sems.at[p - 1],
                recv_sem=recv_sems.at[p - 1],
                device_id=(my,),
                device_id_type=pl.DeviceIdType.MESH,
            )
            recv.wait_recv()
            out_ref[pl.ds(src_id * m_per, m_per), :] = comm_ref[p - 1]

        for rdma in sends:
            rdma.wait_send()

    return pl.pallas_call(
        body,
        out_shape=jax.ShapeDtypeStruct((N_DEV * m_per, n_per), jnp.float32),
        in_specs=[
            pl.BlockSpec(memory_space=pltpu.VMEM),
            pl.BlockSpec(memory_space=pltpu.VMEM),
            pl.BlockSpec(memory_space=pltpu.SMEM),
            pl.BlockSpec(memory_space=pltpu.SMEM),
        ],
        out_specs=pl.BlockSpec(memory_space=pltpu.VMEM),
        scratch_shapes=[
            pltpu.VMEM((N_DEV, m_per, n_per), jnp.float32),
            pltpu.VMEM((N_DEV - 1, m_per, n_per), jnp.float32),
            pltpu.SemaphoreType.DMA((N_DEV - 1,)),
            pltpu.SemaphoreType.DMA((N_DEV - 1,)),
        ],
        compiler_params=pltpu.CompilerParams(collective_id=0),
    )(x, w_mat, scale_x, scale_w)


# baseline (device time: 65349 ns/iter reference)
import jax
import jax.numpy as jnp
from jax import lax
from jax.experimental import pallas as pl
from jax.experimental.pallas import tpu as pltpu

N_DEV = 4
K_STEPS = 4


def kernel(x, w_mat, scale_x, scale_w):
    m_per, k = x.shape
    _, n = w_mat.shape
    n_per = n // N_DEV
    k_chunk = k // K_STEPS

    def body(x_ref, w_ref, sx_ref, sw_ref, out_ref,
             acc_ref, y_ref, comm_ref, send_sems, recv_sems):
        my = lax.axis_index("i")
        kc = pl.program_id(0)

        @pl.when(kc == 0)
        def _barrier():
            barrier_sem = pltpu.get_barrier_semaphore()
            for p in range(1, N_DEV):
                pl.semaphore_signal(
                    barrier_sem, inc=1,
                    device_id=((my + p) % N_DEV,),
                    device_id_type=pl.DeviceIdType.MESH,
                )
            pl.semaphore_wait(barrier_sem, N_DEV - 1)

        xb = x_ref[...].astype(jnp.bfloat16)
        wb = w_ref[...].astype(jnp.bfloat16)
        prod = jnp.dot(xb, wb, preferred_element_type=jnp.float32)

        @pl.when(kc == 0)
        def _init():
            acc_ref[...] = prod

        @pl.when(kc > 0)
        def _accum():
            acc_ref[...] += prod

        @pl.when(kc == K_STEPS - 1)
        def _finish():
            sc = sx_ref[0] * sw_ref[0]
            for d in range(N_DEV):
                yd = acc_ref[:, d * n_per:(d + 1) * n_per] * sc
                yd = yd / (1.0 + jnp.exp(-jnp.clip(yd, -60.0, 60.0)))
                y_ref[d] = yd.astype(jnp.bfloat16)

            out_ref[pl.ds(my * m_per, m_per), :] = y_ref[my].astype(jnp.float32)

            sends = []
            for p in range(1, N_DEV):
                dst = (my + p) % N_DEV
                rdma = pltpu.make_async_remote_copy(
                    src_ref=y_ref.at[dst],
                    dst_ref=comm_ref.at[p - 1],
                    send_sem=send_sems.at[p - 1],
                    recv_sem=recv_sems.at[p - 1],
                    device_id=(dst,),
                    device_id_type=pl.DeviceIdType.MESH,
                )
                rdma.start()
                sends.append(rdma)

            for p in range(1, N_DEV):
                src_id = (my - p) % N_DEV
                recv = pltpu.make_async_remote_copy(
                    src_ref=y_ref.at[0],
                    dst_ref=comm_ref.at[p - 1],
                    send_sem=send_sems.at[p - 1],
                    recv_sem=recv_sems.at[p - 1],
                    device_id=(my,),
                    device_id_type=pl.DeviceIdType.MESH,
                )
                recv.wait_recv()
                out_ref[pl.ds(src_id * m_per, m_per), :] = (
                    comm_ref[p - 1].astype(jnp.float32))

            for rdma in sends:
                rdma.wait_send()

    grid = (K_STEPS,)
    return pl.pallas_call(
        body,
        grid=grid,
        out_shape=jax.ShapeDtypeStruct((N_DEV * m_per, n_per), jnp.float32),
        in_specs=[
            pl.BlockSpec((m_per, k_chunk), lambda kc: (0, kc)),
            pl.BlockSpec((k_chunk, n), lambda kc: (kc, 0)),
            pl.BlockSpec(memory_space=pltpu.SMEM),
            pl.BlockSpec(memory_space=pltpu.SMEM),
        ],
        out_specs=pl.BlockSpec((N_DEV * m_per, n_per), lambda kc: (0, 0)),
        scratch_shapes=[
            pltpu.VMEM((m_per, n), jnp.float32),
            pltpu.VMEM((N_DEV, m_per, n_per), jnp.bfloat16),
            pltpu.VMEM((N_DEV - 1, m_per, n_per), jnp.bfloat16),
            pltpu.SemaphoreType.DMA((N_DEV - 1,)),
            pltpu.SemaphoreType.DMA((N_DEV - 1,)),
        ],
        compiler_params=pltpu.CompilerParams(
            collective_id=0,
            vmem_limit_bytes=60 * 1024 * 1024,
        ),
    )(x, w_mat, scale_x, scale_w)


# device time: 58098 ns/iter; 1.1248x vs baseline; 1.1248x over previous
import jax
import jax.numpy as jnp
from jax import lax
from jax.experimental import pallas as pl
from jax.experimental.pallas import tpu as pltpu

N_DEV = 4


def kernel(x, w_mat, scale_x, scale_w):
    m_per, k = x.shape
    _, n = w_mat.shape
    n_per = n // N_DEV

    def body(perm_ref, x_ref, w_ref, sx_ref, sw_ref, out_ref,
             xb_ref, y_ref, comm_ref, send_sems, recv_sems):
        my = lax.axis_index("i")
        kc = pl.program_id(0)
        d = perm_ref[kc]

        @pl.when(kc == 0)
        def _prologue():
            barrier_sem = pltpu.get_barrier_semaphore()
            for p in range(1, N_DEV):
                pl.semaphore_signal(
                    barrier_sem, inc=1,
                    device_id=((my + p) % N_DEV,),
                    device_id_type=pl.DeviceIdType.MESH,
                )
            pl.semaphore_wait(barrier_sem, N_DEV - 1)
            xb_ref[...] = x_ref[...].astype(jnp.bfloat16)

        wb = w_ref[...].astype(jnp.bfloat16)
        acc = jnp.dot(xb_ref[...], wb, preferred_element_type=jnp.float32)
        sc = sx_ref[0] * sw_ref[0]
        yd = acc * sc
        yd = yd / (1.0 + jnp.exp(-jnp.clip(yd, -60.0, 60.0)))
        y_ref[d] = yd.astype(jnp.bfloat16)

        @pl.when(kc < N_DEV - 1)
        def _send():
            rdma = pltpu.make_async_remote_copy(
                src_ref=y_ref.at[d],
                dst_ref=comm_ref.at[my],
                send_sem=send_sems.at[kc],
                recv_sem=recv_sems.at[my],
                device_id=(d,),
                device_id_type=pl.DeviceIdType.MESH,
            )
            rdma.start()

        @pl.when(kc == N_DEV - 1)
        def _finish():
            out_ref[pl.ds(my * m_per, m_per), :] = y_ref[my].astype(jnp.float32)

            for p in range(1, N_DEV):
                src_id = (my - p) % N_DEV
                recv = pltpu.make_async_remote_copy(
                    src_ref=y_ref.at[0],
                    dst_ref=comm_ref.at[src_id],
                    send_sem=send_sems.at[0],
                    recv_sem=recv_sems.at[src_id],
                    device_id=(my,),
                    device_id_type=pl.DeviceIdType.MESH,
                )
                recv.wait_recv()
                out_ref[pl.ds(src_id * m_per, m_per), :] = (
                    comm_ref[src_id].astype(jnp.float32))

            for s in range(N_DEV - 1):
                done = pltpu.make_async_remote_copy(
                    src_ref=y_ref.at[0],
                    dst_ref=comm_ref.at[0],
                    send_sem=send_sems.at[s],
                    recv_sem=recv_sems.at[0],
                    device_id=(my,),
                    device_id_type=pl.DeviceIdType.MESH,
                )
                done.wait_send()

    perm = (lax.axis_index("i") + 1
            + jnp.arange(N_DEV, dtype=jnp.int32)) % N_DEV

    grid_spec = pltpu.PrefetchScalarGridSpec(
        num_scalar_prefetch=1,
        grid=(N_DEV,),
        in_specs=[
            pl.BlockSpec((m_per, k), lambda kc, perm: (0, 0)),
            pl.BlockSpec((k, n_per), lambda kc, perm: (0, perm[kc])),
            pl.BlockSpec(memory_space=pltpu.SMEM),
            pl.BlockSpec(memory_space=pltpu.SMEM),
        ],
        out_specs=pl.BlockSpec((N_DEV * m_per, n_per), lambda kc, perm: (0, 0)),
        scratch_shapes=[
            pltpu.VMEM((m_per, k), jnp.bfloat16),
            pltpu.VMEM((N_DEV, m_per, n_per), jnp.bfloat16),
            pltpu.VMEM((N_DEV, m_per, n_per), jnp.bfloat16),
            pltpu.SemaphoreType.DMA((N_DEV - 1,)),
            pltpu.SemaphoreType.DMA((N_DEV,)),
        ],
    )
    return pl.pallas_call(
        body,
        grid_spec=grid_spec,
        out_shape=jax.ShapeDtypeStruct((N_DEV * m_per, n_per), jnp.float32),
        compiler_params=pltpu.CompilerParams(
            collective_id=0,
            vmem_limit_bytes=60 * 1024 * 1024,
        ),
    )(perm, x, w_mat, scale_x, scale_w)


# device time: 50017 ns/iter; 1.3065x vs baseline; 1.1616x over previous
import jax
import jax.numpy as jnp
from jax import lax
from jax.experimental import pallas as pl
from jax.experimental.pallas import tpu as pltpu

N_DEV = 4
N_HB = 2 * N_DEV


def kernel(x, w_mat, scale_x, scale_w):
    m_per, k = x.shape
    _, n = w_mat.shape
    n_per = n // N_DEV
    n_half = n_per // 2

    def body(perm_ref, x_ref, w_ref, sx_ref, sw_ref, out_ref,
             xb_ref, y_ref, comm_ref, send_sems, recv_sems):
        my = lax.axis_index("i")
        kc = pl.program_id(0)
        phb = perm_ref[kc]
        dchip = phb // 2
        half = phb % 2

        @pl.when(kc == 0)
        def _prologue():
            barrier_sem = pltpu.get_barrier_semaphore()
            for p in range(1, N_DEV):
                pl.semaphore_signal(
                    barrier_sem, inc=1,
                    device_id=((my + p) % N_DEV,),
                    device_id_type=pl.DeviceIdType.MESH,
                )
            pl.semaphore_wait(barrier_sem, N_DEV - 1)
            xb_ref[...] = x_ref[...].astype(jnp.bfloat16)

        wb = w_ref[...].astype(jnp.bfloat16)
        acc = jnp.dot(xb_ref[...], wb, preferred_element_type=jnp.float32)
        sc = sx_ref[0] * sw_ref[0]
        yd = acc * sc
        yd = yd / (1.0 + jnp.exp(-jnp.clip(yd, -60.0, 60.0)))
        y_ref[phb] = yd.astype(jnp.bfloat16)

        @pl.when(kc < N_HB - 2)
        def _send():
            rdma = pltpu.make_async_remote_copy(
                src_ref=y_ref.at[phb],
                dst_ref=comm_ref.at[my * 2 + half],
                send_sem=send_sems.at[kc],
                recv_sem=recv_sems.at[my * 2 + half],
                device_id=(dchip,),
                device_id_type=pl.DeviceIdType.MESH,
            )
            rdma.start()

        @pl.when(kc == N_HB - 1)
        def _finish():
            for h in range(2):
                out_ref[pl.ds(my * m_per, m_per),
                        h * n_half:(h + 1) * n_half] = (
                    y_ref[my * 2 + h].astype(jnp.float32))

            for p in (2, 1, 3):
                src_id = (my - p) % N_DEV
                for h in range(2):
                    recv = pltpu.make_async_remote_copy(
                        src_ref=y_ref.at[0],
                        dst_ref=comm_ref.at[src_id * 2 + h],
                        send_sem=send_sems.at[0],
                        recv_sem=recv_sems.at[src_id * 2 + h],
                        device_id=(my,),
                        device_id_type=pl.DeviceIdType.MESH,
                    )
                    recv.wait_recv()
                    out_ref[pl.ds(src_id * m_per, m_per),
                            h * n_half:(h + 1) * n_half] = (
                        comm_ref[src_id * 2 + h].astype(jnp.float32))

            for s in range(N_HB - 2):
                done = pltpu.make_async_remote_copy(
                    src_ref=y_ref.at[0],
                    dst_ref=comm_ref.at[0],
                    send_sem=send_sems.at[s],
                    recv_sem=recv_sems.at[0],
                    device_id=(my,),
                    device_id_type=pl.DeviceIdType.MESH,
                )
                done.wait_send()

    my_idx = lax.axis_index("i")
    block_order = (my_idx + jnp.array([2, 1, 3, 0], dtype=jnp.int32)) % N_DEV
    perm = (block_order[:, None] * 2
            + jnp.arange(2, dtype=jnp.int32)[None, :]).reshape(N_HB)

    grid_spec = pltpu.PrefetchScalarGridSpec(
        num_scalar_prefetch=1,
        grid=(N_HB,),
        in_specs=[
            pl.BlockSpec((m_per, k), lambda kc, perm: (0, 0)),
            pl.BlockSpec((k, n_half), lambda kc, perm: (0, perm[kc])),
            pl.BlockSpec(memory_space=pltpu.SMEM),
            pl.BlockSpec(memory_space=pltpu.SMEM),
        ],
        out_specs=pl.BlockSpec((N_DEV * m_per, n_per), lambda kc, perm: (0, 0)),
        scratch_shapes=[
            pltpu.VMEM((m_per, k), jnp.bfloat16),
            pltpu.VMEM((N_HB, m_per, n_half), jnp.bfloat16),
            pltpu.VMEM((N_HB, m_per, n_half), jnp.bfloat16),
            pltpu.SemaphoreType.DMA((N_HB - 2,)),
            pltpu.SemaphoreType.DMA((N_HB,)),
        ],
    )
    return pl.pallas_call(
        body,
        grid_spec=grid_spec,
        out_shape=jax.ShapeDtypeStruct((N_DEV * m_per, n_per), jnp.float32),
        compiler_params=pltpu.CompilerParams(
            collective_id=0,
            vmem_limit_bytes=60 * 1024 * 1024,
        ),
    )(perm, x, w_mat, scale_x, scale_w)


# device time: 49986 ns/iter; 1.3073x vs baseline; 1.0006x over previous
import jax
import jax.numpy as jnp
from jax import lax
from jax.experimental import pallas as pl
from jax.experimental.pallas import tpu as pltpu

N_DEV = 4
N_HB = 2 * N_DEV


def kernel(x, w_mat, scale_x, scale_w):
    m_per, k = x.shape
    _, n = w_mat.shape
    n_per = n // N_DEV
    n_half = n_per // 2

    def body(perm_ref, x_ref, w_ref, sx_ref, sw_ref, out_ref,
             xb_ref, y_ref, comm_ref, send_sems, recv_sems):
        my = lax.axis_index("i")
        kc = pl.program_id(0)
        phb = perm_ref[kc]
        dchip = phb // 2
        half = phb % 2

        @pl.when(kc == 0)
        def _prologue():
            barrier_sem = pltpu.get_barrier_semaphore()
            for p in range(1, N_DEV):
                pl.semaphore_signal(
                    barrier_sem, inc=1,
                    device_id=((my + p) % N_DEV,),
                    device_id_type=pl.DeviceIdType.MESH,
                )
            pl.semaphore_wait(barrier_sem, N_DEV - 1)
            xb_ref[...] = x_ref[...].astype(jnp.bfloat16)

        K_SUB = 4
        k_sub = k // K_SUB
        acc = jnp.zeros((m_per, n_half), jnp.float32)
        for j in range(K_SUB):
            wbj = w_ref[j * k_sub:(j + 1) * k_sub, :].astype(jnp.bfloat16)
            acc += jnp.dot(xb_ref[:, j * k_sub:(j + 1) * k_sub], wbj,
                           preferred_element_type=jnp.float32)
        sc = sx_ref[0] * sw_ref[0]
        yd = acc * sc
        yd = yd / (1.0 + jnp.exp(-jnp.clip(yd, -60.0, 60.0)))

        @pl.when(kc >= N_HB - 2)
        def _own_store():
            out_ref[pl.ds(my * m_per, m_per),
                    pl.ds(half * n_half, n_half)] = yd

        @pl.when(kc < N_HB - 2)
        def _stage():
            y_ref[phb] = yd.astype(jnp.bfloat16)

        @pl.when(kc < N_HB - 2)
        def _send():
            rdma = pltpu.make_async_remote_copy(
                src_ref=y_ref.at[phb],
                dst_ref=comm_ref.at[my * 2 + half],
                send_sem=send_sems.at[kc],
                recv_sem=recv_sems.at[my * 2 + half],
                device_id=(dchip,),
                device_id_type=pl.DeviceIdType.MESH,
            )
            rdma.start()

        def _recv_from(p):
            src_id = (my - p) % N_DEV
            for h in range(2):
                recv = pltpu.make_async_remote_copy(
                    src_ref=y_ref.at[0],
                    dst_ref=comm_ref.at[src_id * 2 + h],
                    send_sem=send_sems.at[0],
                    recv_sem=recv_sems.at[src_id * 2 + h],
                    device_id=(my,),
                    device_id_type=pl.DeviceIdType.MESH,
                )
                recv.wait_recv()
                out_ref[pl.ds(src_id * m_per, m_per),
                        h * n_half:(h + 1) * n_half] = (
                    comm_ref[src_id * 2 + h].astype(jnp.float32))

        @pl.when(kc == N_HB - 2)
        def _early_recv():
            _recv_from(2)

        @pl.when(kc == N_HB - 1)
        def _finish():
            for p in (1, 3):
                _recv_from(p)

            for s in range(N_HB - 2):
                done = pltpu.make_async_remote_copy(
                    src_ref=y_ref.at[0],
                    dst_ref=comm_ref.at[0],
                    send_sem=send_sems.at[s],
                    recv_sem=recv_sems.at[0],
                    device_id=(my,),
                    device_id_type=pl.DeviceIdType.MESH,
                )
                done.wait_send()

    my_idx = lax.axis_index("i")
    block_order = (my_idx + jnp.array([2, 1, 3, 0], dtype=jnp.int32)) % N_DEV
    perm = (block_order[:, None] * 2
            + jnp.arange(2, dtype=jnp.int32)[None, :]).reshape(N_HB)

    grid_spec = pltpu.PrefetchScalarGridSpec(
        num_scalar_prefetch=1,
        grid=(N_HB,),
        in_specs=[
            pl.BlockSpec((m_per, k), lambda kc, perm: (0, 0)),
            pl.BlockSpec((k, n_half), lambda kc, perm: (0, perm[kc])),
            pl.BlockSpec(memory_space=pltpu.SMEM),
            pl.BlockSpec(memory_space=pltpu.SMEM),
        ],
        out_specs=pl.BlockSpec((N_DEV * m_per, n_per), lambda kc, perm: (0, 0)),
        scratch_shapes=[
            pltpu.VMEM((m_per, k), jnp.bfloat16),
            pltpu.VMEM((N_HB, m_per, n_half), jnp.bfloat16),
            pltpu.VMEM((N_HB, m_per, n_half), jnp.bfloat16),
            pltpu.SemaphoreType.DMA((N_HB - 2,)),
            pltpu.SemaphoreType.DMA((N_HB,)),
        ],
    )
    return pl.pallas_call(
        body,
        grid_spec=grid_spec,
        out_shape=jax.ShapeDtypeStruct((N_DEV * m_per, n_per), jnp.float32),
        compiler_params=pltpu.CompilerParams(
            collective_id=0,
            vmem_limit_bytes=60 * 1024 * 1024,
        ),
    )(perm, x, w_mat, scale_x, scale_w)


# device time: 49021 ns/iter; 1.3331x vs baseline; 1.0197x over previous
import jax
import jax.numpy as jnp
from jax import lax
from jax.experimental import pallas as pl
from jax.experimental.pallas import tpu as pltpu

N_DEV = 4
N_HB = 2 * N_DEV


def kernel(x, w_mat, scale_x, scale_w):
    m_per, k = x.shape
    _, n = w_mat.shape
    n_per = n // N_DEV
    n_half = n_per // 2

    def body(perm_ref, x_ref, w_ref, sx_ref, sw_ref, out_ref,
             xb_ref, y_ref, comm_ref, send_sems, recv_sems):
        my = lax.axis_index("i")
        kc = pl.program_id(0)
        phb = perm_ref[kc]
        dchip = phb // 2
        half = phb % 2

        @pl.when(kc == 0)
        def _prologue():
            barrier_sem = pltpu.get_barrier_semaphore()
            for p in range(1, N_DEV):
                pl.semaphore_signal(
                    barrier_sem, inc=1,
                    device_id=((my + p) % N_DEV,),
                    device_id_type=pl.DeviceIdType.MESH,
                )
            pl.semaphore_wait(barrier_sem, N_DEV - 1)
            xb_ref[...] = x_ref[...].astype(jnp.float8_e5m2)

        wb = w_ref[...].astype(jnp.float8_e5m2)
        acc = jnp.dot(xb_ref[...], wb, preferred_element_type=jnp.float32)
        sc = sx_ref[0] * sw_ref[0]
        yd = acc * sc
        yd = yd / (1.0 + jnp.exp(-jnp.clip(yd, -60.0, 60.0)))

        @pl.when(kc >= N_HB - 2)
        def _own_store():
            out_ref[pl.ds(my * m_per, m_per),
                    pl.ds(half * n_half, n_half)] = yd

        @pl.when(kc < N_HB - 2)
        def _stage():
            y_ref[phb] = yd.astype(jnp.bfloat16)

        @pl.when(kc < N_HB - 2)
        def _send():
            rdma = pltpu.make_async_remote_copy(
                src_ref=y_ref.at[phb],
                dst_ref=comm_ref.at[my * 2 + half],
                send_sem=send_sems.at[kc],
                recv_sem=recv_sems.at[my * 2 + half],
                device_id=(dchip,),
                device_id_type=pl.DeviceIdType.MESH,
            )
            rdma.start()

        def _recv_from(p):
            src_id = (my - p) % N_DEV
            for h in range(2):
                recv = pltpu.make_async_remote_copy(
                    src_ref=y_ref.at[0],
                    dst_ref=comm_ref.at[src_id * 2 + h],
                    send_sem=send_sems.at[0],
                    recv_sem=recv_sems.at[src_id * 2 + h],
                    device_id=(my,),
                    device_id_type=pl.DeviceIdType.MESH,
                )
                recv.wait_recv()
                out_ref[pl.ds(src_id * m_per, m_per),
                        h * n_half:(h + 1) * n_half] = (
                    comm_ref[src_id * 2 + h].astype(jnp.float32))

        @pl.when(kc == N_HB - 2)
        def _early_recv():
            _recv_from(2)

        @pl.when(kc == N_HB - 1)
        def _finish():
            for p in (1, 3):
                _recv_from(p)

            for s in range(N_HB - 2):
                done = pltpu.make_async_remote_copy(
                    src_ref=y_ref.at[0],
                    dst_ref=comm_ref.at[0],
                    send_sem=send_sems.at[s],
                    recv_sem=recv_sems.at[0],
                    device_id=(my,),
                    device_id_type=pl.DeviceIdType.MESH,
                )
                done.wait_send()

    my_idx = lax.axis_index("i")
    block_order = (my_idx + jnp.array([2, 1, 3, 0], dtype=jnp.int32)) % N_DEV
    perm = (block_order[:, None] * 2
            + jnp.arange(2, dtype=jnp.int32)[None, :]).reshape(N_HB)

    grid_spec = pltpu.PrefetchScalarGridSpec(
        num_scalar_prefetch=1,
        grid=(N_HB,),
        in_specs=[
            pl.BlockSpec((m_per, k), lambda kc, perm: (0, 0)),
            pl.BlockSpec((k, n_half), lambda kc, perm: (0, perm[kc])),
            pl.BlockSpec(memory_space=pltpu.SMEM),
            pl.BlockSpec(memory_space=pltpu.SMEM),
        ],
        out_specs=pl.BlockSpec((N_DEV * m_per, n_per), lambda kc, perm: (0, 0)),
        scratch_shapes=[
            pltpu.VMEM((m_per, k), jnp.float8_e5m2),
            pltpu.VMEM((N_HB, m_per, n_half), jnp.bfloat16),
            pltpu.VMEM((N_HB, m_per, n_half), jnp.bfloat16),
            pltpu.SemaphoreType.DMA((N_HB - 2,)),
            pltpu.SemaphoreType.DMA((N_HB,)),
        ],
    )
    return pl.pallas_call(
        body,
        grid_spec=grid_spec,
        out_shape=jax.ShapeDtypeStruct((N_DEV * m_per, n_per), jnp.float32),
        compiler_params=pltpu.CompilerParams(
            collective_id=0,
            vmem_limit_bytes=60 * 1024 * 1024,
        ),
    )(perm, x, w_mat, scale_x, scale_w)


# device time: 40879 ns/iter; 1.5986x vs baseline; 1.1992x over previous
import jax
import jax.numpy as jnp
from jax import lax
from jax.experimental import pallas as pl
from jax.experimental.pallas import tpu as pltpu

N_DEV = 4
N_HB = 2 * N_DEV


def kernel(x, w_mat, scale_x, scale_w):
    m_per, k = x.shape
    _, n = w_mat.shape
    n_per = n // N_DEV
    n_half = n_per // 2

    def body(perm_ref, x_ref, w_ref, sx_ref, sw_ref, out_ref,
             xb_ref, y_ref, ysc_ref, comm_ref, csc_ref,
             send_sems, recv_sems, ssc_send, ssc_recv):
        my = lax.axis_index("i")
        kc = pl.program_id(0)
        phb = perm_ref[kc]
        dchip = phb // 2
        half = phb % 2

        @pl.when(kc == 0)
        def _prologue():
            barrier_sem = pltpu.get_barrier_semaphore()
            for p in range(1, N_DEV):
                pl.semaphore_signal(
                    barrier_sem, inc=1,
                    device_id=((my + p) % N_DEV,),
                    device_id_type=pl.DeviceIdType.MESH,
                )
            pl.semaphore_wait(barrier_sem, N_DEV - 1)
            xb_ref[...] = x_ref[...].astype(jnp.float8_e5m2)

        wb = w_ref[...].astype(jnp.float8_e5m2)
        acc = jnp.dot(xb_ref[...], wb, preferred_element_type=jnp.float32)
        sc = sx_ref[0] * sw_ref[0]
        yd = acc * sc
        yd = yd / (1.0 + jnp.exp(-jnp.clip(yd, -60.0, 60.0)))

        @pl.when(kc >= N_HB - 2)
        def _own_store():
            out_ref[pl.ds(my * m_per, m_per),
                    pl.ds(half * n_half, n_half)] = yd

        @pl.when(kc < N_HB - 2)
        def _stage():
            s = jnp.max(jnp.abs(yd)) + 1e-20
            y_ref[phb] = jnp.round(yd * (127.0 / s)).astype(jnp.int8)
            ysc_ref[phb] = jnp.broadcast_to(s * (1.0 / 127.0), (8, 128))

        @pl.when(kc < N_HB - 2)
        def _send():
            rdma = pltpu.make_async_remote_copy(
                src_ref=y_ref.at[phb],
                dst_ref=comm_ref.at[my * 2 + half],
                send_sem=send_sems.at[kc],
                recv_sem=recv_sems.at[my * 2 + half],
                device_id=(dchip,),
                device_id_type=pl.DeviceIdType.MESH,
            )
            rdma.start()
            rs = pltpu.make_async_remote_copy(
                src_ref=ysc_ref.at[phb],
                dst_ref=csc_ref.at[my * 2 + half],
                send_sem=ssc_send.at[kc],
                recv_sem=ssc_recv.at[my * 2 + half],
                device_id=(dchip,),
                device_id_type=pl.DeviceIdType.MESH,
            )
            rs.start()

        def _recv_from(p):
            src_id = (my - p) % N_DEV
            for h in range(2):
                idx = src_id * 2 + h
                recv = pltpu.make_async_remote_copy(
                    src_ref=y_ref.at[0],
                    dst_ref=comm_ref.at[idx],
                    send_sem=send_sems.at[0],
                    recv_sem=recv_sems.at[idx],
                    device_id=(my,),
                    device_id_type=pl.DeviceIdType.MESH,
                )
                recv.wait_recv()
                rsc = pltpu.make_async_remote_copy(
                    src_ref=ysc_ref.at[0],
                    dst_ref=csc_ref.at[idx],
                    send_sem=ssc_send.at[0],
                    recv_sem=ssc_recv.at[idx],
                    device_id=(my,),
                    device_id_type=pl.DeviceIdType.MESH,
                )
                rsc.wait_recv()
                out_ref[pl.ds(src_id * m_per, m_per),
                        h * n_half:(h + 1) * n_half] = (
                    comm_ref[idx].astype(jnp.float32) * csc_ref[idx, 0, 0])

        @pl.when(kc == N_HB - 2)
        def _early_recv():
            _recv_from(2)

        @pl.when(kc == N_HB - 1)
        def _finish():
            for p in (1, 3):
                _recv_from(p)

            for s in range(N_HB - 2):
                done = pltpu.make_async_remote_copy(
                    src_ref=y_ref.at[0],
                    dst_ref=comm_ref.at[0],
                    send_sem=send_sems.at[s],
                    recv_sem=recv_sems.at[0],
                    device_id=(my,),
                    device_id_type=pl.DeviceIdType.MESH,
                )
                done.wait_send()
                dsc = pltpu.make_async_remote_copy(
                    src_ref=ysc_ref.at[0],
                    dst_ref=csc_ref.at[0],
                    send_sem=ssc_send.at[s],
                    recv_sem=ssc_recv.at[0],
                    device_id=(my,),
                    device_id_type=pl.DeviceIdType.MESH,
                )
                dsc.wait_send()

    my_idx = lax.axis_index("i")
    block_order = (my_idx + jnp.array([2, 1, 3, 0], dtype=jnp.int32)) % N_DEV
    perm = (block_order[:, None] * 2
            + jnp.arange(2, dtype=jnp.int32)[None, :]).reshape(N_HB)

    grid_spec = pltpu.PrefetchScalarGridSpec(
        num_scalar_prefetch=1,
        grid=(N_HB,),
        in_specs=[
            pl.BlockSpec((m_per, k), lambda kc, perm: (0, 0)),
            pl.BlockSpec((k, n_half), lambda kc, perm: (0, perm[kc])),
            pl.BlockSpec(memory_space=pltpu.SMEM),
            pl.BlockSpec(memory_space=pltpu.SMEM),
        ],
        out_specs=pl.BlockSpec((N_DEV * m_per, n_per), lambda kc, perm: (0, 0)),
        scratch_shapes=[
            pltpu.VMEM((m_per, k), jnp.float8_e5m2),
            pltpu.VMEM((N_HB, m_per, n_half), jnp.int8),
            pltpu.VMEM((N_HB, 8, 128), jnp.float32),
            pltpu.VMEM((N_HB, m_per, n_half), jnp.int8),
            pltpu.VMEM((N_HB, 8, 128), jnp.float32),
            pltpu.SemaphoreType.DMA((N_HB - 2,)),
            pltpu.SemaphoreType.DMA((N_HB,)),
            pltpu.SemaphoreType.DMA((N_HB - 2,)),
            pltpu.SemaphoreType.DMA((N_HB,)),
        ],
    )
    return pl.pallas_call(
        body,
        grid_spec=grid_spec,
        out_shape=jax.ShapeDtypeStruct((N_DEV * m_per, n_per), jnp.float32),
        compiler_params=pltpu.CompilerParams(
            collective_id=0,
            vmem_limit_bytes=60 * 1024 * 1024,
        ),
    )(perm, x, w_mat, scale_x, scale_w)


# device time: 38970 ns/iter; 1.6769x vs baseline; 1.0490x over previous
import jax
import jax.numpy as jnp
from jax import lax
from jax.experimental import pallas as pl
from jax.experimental.pallas import tpu as pltpu

N_DEV = 4
N_HB = 2 * N_DEV


def kernel(x, w_mat, scale_x, scale_w):
    m_per, k = x.shape
    _, n = w_mat.shape
    n_per = n // N_DEV
    n_half = n_per // 2

    def body(perm_ref, x_ref, w_ref, sx_ref, sw_ref, out_ref,
             xb_ref, y_ref, ysc_ref, comm_ref, csc_ref, ostage_ref,
             send_sems, recv_sems, ssc_send, ssc_recv, out_sems):
        my = lax.axis_index("i")
        kc = pl.program_id(0)
        phb = perm_ref[kc]
        dchip = phb // 2
        half = phb % 2

        @pl.when(kc == 0)
        def _prologue():
            barrier_sem = pltpu.get_barrier_semaphore()
            for p in range(1, N_DEV):
                pl.semaphore_signal(
                    barrier_sem, inc=1,
                    device_id=((my + p) % N_DEV,),
                    device_id_type=pl.DeviceIdType.MESH,
                )
            pl.semaphore_wait(barrier_sem, N_DEV - 1)
            xb_ref[...] = x_ref[...].astype(jnp.float8_e5m2)

        wb = w_ref[...].astype(jnp.float8_e5m2)
        acc = jnp.dot(xb_ref[...], wb, preferred_element_type=jnp.float32)
        sc = sx_ref[0] * sw_ref[0]
        yd = acc * sc
        yd = yd / (1.0 + jnp.exp(-jnp.clip(yd, -60.0, 60.0)))

        @pl.when(kc >= N_HB - 2)
        def _own_store():
            ostage_ref[pl.ds(my * m_per, m_per),
                       pl.ds(half * n_half, n_half)] = yd
            cp = pltpu.make_async_copy(
                ostage_ref.at[pl.ds(my * m_per, m_per),
                              pl.ds(half * n_half, n_half)],
                out_ref.at[pl.ds(my * m_per, m_per),
                           pl.ds(half * n_half, n_half)],
                out_sems.at[my * 2 + half],
            )
            cp.start()

        @pl.when(kc < N_HB - 2)
        def _stage():
            s = jnp.max(jnp.abs(yd)) + 1e-20
            y_ref[phb] = jnp.round(yd * (127.0 / s)).astype(jnp.int8)
            ysc_ref[phb] = jnp.broadcast_to(s * (1.0 / 127.0), (8, 128))

        @pl.when(kc < N_HB - 2)
        def _send():
            rdma = pltpu.make_async_remote_copy(
                src_ref=y_ref.at[phb],
                dst_ref=comm_ref.at[my * 2 + half],
                send_sem=send_sems.at[kc],
                recv_sem=recv_sems.at[my * 2 + half],
                device_id=(dchip,),
                device_id_type=pl.DeviceIdType.MESH,
            )
            rdma.start()
            rs = pltpu.make_async_remote_copy(
                src_ref=ysc_ref.at[phb],
                dst_ref=csc_ref.at[my * 2 + half],
                send_sem=ssc_send.at[kc],
                recv_sem=ssc_recv.at[my * 2 + half],
                device_id=(dchip,),
                device_id_type=pl.DeviceIdType.MESH,
            )
            rs.start()

        def _recv_from(p):
            src_id = (my - p) % N_DEV
            for h in range(2):
                idx = src_id * 2 + h
                recv = pltpu.make_async_remote_copy(
                    src_ref=y_ref.at[0],
                    dst_ref=comm_ref.at[idx],
                    send_sem=send_sems.at[0],
                    recv_sem=recv_sems.at[idx],
                    device_id=(my,),
                    device_id_type=pl.DeviceIdType.MESH,
                )
                recv.wait_recv()
                rsc = pltpu.make_async_remote_copy(
                    src_ref=ysc_ref.at[0],
                    dst_ref=csc_ref.at[idx],
                    send_sem=ssc_send.at[0],
                    recv_sem=ssc_recv.at[idx],
                    device_id=(my,),
                    device_id_type=pl.DeviceIdType.MESH,
                )
                rsc.wait_recv()
                ostage_ref[pl.ds(src_id * m_per, m_per),
                           h * n_half:(h + 1) * n_half] = (
                    comm_ref[idx].astype(jnp.float32) * csc_ref[idx, 0, 0])
                cp = pltpu.make_async_copy(
                    ostage_ref.at[pl.ds(src_id * m_per, m_per),
                                  h * n_half:(h + 1) * n_half],
                    out_ref.at[pl.ds(src_id * m_per, m_per),
                               h * n_half:(h + 1) * n_half],
                    out_sems.at[idx],
                )
                cp.start()

        @pl.when(kc == N_HB - 2)
        def _early_recv():
            _recv_from(2)

        @pl.when(kc == N_HB - 1)
        def _finish():
            for p in (1, 3):
                _recv_from(p)

            for s in range(N_HB - 2):
                done = pltpu.make_async_remote_copy(
                    src_ref=y_ref.at[0],
                    dst_ref=comm_ref.at[0],
                    send_sem=send_sems.at[s],
                    recv_sem=recv_sems.at[0],
                    device_id=(my,),
                    device_id_type=pl.DeviceIdType.MESH,
                )
                done.wait_send()
                dsc = pltpu.make_async_remote_copy(
                    src_ref=ysc_ref.at[0],
                    dst_ref=csc_ref.at[0],
                    send_sem=ssc_send.at[s],
                    recv_sem=ssc_recv.at[0],
                    device_id=(my,),
                    device_id_type=pl.DeviceIdType.MESH,
                )
                dsc.wait_send()

            for b in range(N_DEV):
                for h in range(2):
                    cp = pltpu.make_async_copy(
                        ostage_ref.at[pl.ds(b * m_per, m_per),
                                      h * n_half:(h + 1) * n_half],
                        out_ref.at[pl.ds(b * m_per, m_per),
                                   h * n_half:(h + 1) * n_half],
                        out_sems.at[b * 2 + h],
                    )
                    cp.wait()

    my_idx = lax.axis_index("i")
    block_order = (my_idx + jnp.array([2, 1, 3, 0], dtype=jnp.int32)) % N_DEV
    perm = (block_order[:, None] * 2
            + jnp.arange(2, dtype=jnp.int32)[None, :]).reshape(N_HB)

    grid_spec = pltpu.PrefetchScalarGridSpec(
        num_scalar_prefetch=1,
        grid=(N_HB,),
        in_specs=[
            pl.BlockSpec((m_per, k), lambda kc, perm: (0, 0)),
            pl.BlockSpec((k, n_half), lambda kc, perm: (0, perm[kc])),
            pl.BlockSpec(memory_space=pltpu.SMEM),
            pl.BlockSpec(memory_space=pltpu.SMEM),
        ],
        out_specs=pl.BlockSpec(memory_space=pl.ANY),
        scratch_shapes=[
            pltpu.VMEM((m_per, k), jnp.float8_e5m2),
            pltpu.VMEM((N_HB, m_per, n_half), jnp.int8),
            pltpu.VMEM((N_HB, 8, 128), jnp.float32),
            pltpu.VMEM((N_HB, m_per, n_half), jnp.int8),
            pltpu.VMEM((N_HB, 8, 128), jnp.float32),
            pltpu.VMEM((N_DEV * m_per, n_per), jnp.float32),
            pltpu.SemaphoreType.DMA((N_HB - 2,)),
            pltpu.SemaphoreType.DMA((N_HB,)),
            pltpu.SemaphoreType.DMA((N_HB - 2,)),
            pltpu.SemaphoreType.DMA((N_HB,)),
            pltpu.SemaphoreType.DMA((N_HB,)),
        ],
    )
    return pl.pallas_call(
        body,
        grid_spec=grid_spec,
        out_shape=jax.ShapeDtypeStruct((N_DEV * m_per, n_per), jnp.float32),
        compiler_params=pltpu.CompilerParams(
            collective_id=0,
            vmem_limit_bytes=60 * 1024 * 1024,
        ),
    )(perm, x, w_mat, scale_x, scale_w)
